# baseline (speedup 1.0000x reference)
import functools

import jax
import jax.numpy as jnp
from jax import lax
from jax.experimental import pallas as pl
from jax.experimental.pallas import tpu as pltpu
from jax.experimental.pallas import tpu_sc as plsc

BATCH = 16384
EMBED_DIM = 32
HIDDEN_DIM = 64
EPS = 1e-5

SC_HALF = 8192
TC_HALF = BATCH - SC_HALF

NUM_CORES = 2
NUM_SUBCORES = 16
NUM_WORKERS = NUM_CORES * NUM_SUBCORES
ROWS_PER_WORKER = SC_HALF // NUM_WORKERS
GROUP = 16
NGROUPS = ROWS_PER_WORKER // GROUP
PRIME = 4


def _sc_gather(table, ids2d):
    mesh = plsc.VectorSubcoreMesh(core_axis_name="c", subcore_axis_name="s")

    @functools.partial(
        pl.kernel,
        mesh=mesh,
        out_type=jax.ShapeDtypeStruct((SC_HALF, EMBED_DIM), jnp.float32),
        scratch_types=[
            pltpu.VMEM((ROWS_PER_WORKER,), jnp.int32),
            pltpu.VMEM((ROWS_PER_WORKER, EMBED_DIM), jnp.float32),
            pltpu.SemaphoreType.DMA,
            pltpu.SemaphoreType.DMA,
        ],
    )
    def gather_kernel(table_hbm, idx_hbm, out_hbm, idx_v, rows_v, sem_i, sem):
        wid = lax.axis_index("s") * NUM_CORES + lax.axis_index("c")
        pltpu.async_copy(idx_hbm.at[wid], idx_v, sem_i).wait()

        def fire_group(base):
            v = idx_v[pl.ds(base, GROUP)]
            for j in range(GROUP):
                pltpu.async_copy(
                    table_hbm.at[pl.ds(v[j], 1)], rows_v.at[pl.ds(base + j, 1)], sem
                )

        def wait_group():
            for _ in range(GROUP):
                pltpu.make_async_copy(
                    table_hbm.at[pl.ds(0, 1)], rows_v.at[pl.ds(0, 1)], sem
                ).wait()

        for g in range(PRIME):
            fire_group(g * GROUP)

        @pl.loop(PRIME, NGROUPS)
        def _(g):
            fire_group(g * GROUP)
            wait_group()

        for _ in range(PRIME):
            wait_group()

        pltpu.sync_copy(rows_v, out_hbm.at[pl.ds(wid * ROWS_PER_WORKER, ROWS_PER_WORKER)])

    return gather_kernel(table, ids2d)


def _mlp_math(emb, w, b, g, be):
    h = lax.dot_general(
        emb,
        w,
        (((1,), (0,)), ((), ())),
        precision=lax.Precision.HIGHEST,
        preferred_element_type=jnp.float32,
    )
    h = h + b
    mu = jnp.mean(h, axis=1, keepdims=True)
    var = jnp.mean((h - mu) ** 2, axis=1, keepdims=True)
    hn = (h - mu) * lax.rsqrt(var + EPS)
    return jnp.maximum(hn * g + be, 0.0)


_TCG_BLOCK = 1024


def _tcg_body(ids_ref, table_ref, w_ref, b_ref, g_ref, be_ref, out_ref, emb_v, sem):
    base = pl.program_id(0) * _TCG_BLOCK

    @pl.loop(0, _TCG_BLOCK)
    def _(i):
        r = ids_ref[base + i]
        pltpu.make_async_copy(
            table_ref.at[pl.ds(r, 1)], emb_v.at[pl.ds(i, 1)], sem
        ).start()

    @pl.loop(0, _TCG_BLOCK)
    def _(i):
        pltpu.make_async_copy(
            table_ref.at[pl.ds(0, 1)], emb_v.at[pl.ds(0, 1)], sem
        ).wait()

    out_ref[...] = _mlp_math(emb_v[...], w_ref[...], b_ref[...], g_ref[...], be_ref[...])


def _tc_gather_mlp(ids_tc, table, W, b, gamma, beta):
    grid_spec = pltpu.PrefetchScalarGridSpec(
        num_scalar_prefetch=1,
        grid=(TC_HALF // _TCG_BLOCK,),
        in_specs=[
            pl.BlockSpec(memory_space=pl.ANY),
            pl.BlockSpec((EMBED_DIM, HIDDEN_DIM), lambda i, ids: (0, 0)),
            pl.BlockSpec((1, HIDDEN_DIM), lambda i, ids: (0, 0)),
            pl.BlockSpec((1, HIDDEN_DIM), lambda i, ids: (0, 0)),
            pl.BlockSpec((1, HIDDEN_DIM), lambda i, ids: (0, 0)),
        ],
        out_specs=pl.BlockSpec((_TCG_BLOCK, HIDDEN_DIM), lambda i, ids: (i, 0)),
        scratch_shapes=[
            pltpu.VMEM((_TCG_BLOCK, EMBED_DIM), jnp.float32),
            pltpu.SemaphoreType.DMA,
        ],
    )
    return pl.pallas_call(
        _tcg_body,
        grid_spec=grid_spec,
        out_shape=jax.ShapeDtypeStruct((TC_HALF, HIDDEN_DIM), jnp.float32),
    )(ids_tc, table, W, b, gamma, beta)


_MLP_BLOCK = 2048


def _mlp_body(emb_ref, w_ref, b_ref, g_ref, be_ref, out_ref):
    out_ref[...] = _mlp_math(
        emb_ref[...], w_ref[...], b_ref[...], g_ref[...], be_ref[...]
    )


def _tc_mlp(emb, W, b, gamma, beta):
    n = emb.shape[0]
    return pl.pallas_call(
        _mlp_body,
        grid=(n // _MLP_BLOCK,),
        in_specs=[
            pl.BlockSpec((_MLP_BLOCK, EMBED_DIM), lambda i: (i, 0)),
            pl.BlockSpec((EMBED_DIM, HIDDEN_DIM), lambda i: (0, 0)),
            pl.BlockSpec((1, HIDDEN_DIM), lambda i: (0, 0)),
            pl.BlockSpec((1, HIDDEN_DIM), lambda i: (0, 0)),
            pl.BlockSpec((1, HIDDEN_DIM), lambda i: (0, 0)),
        ],
        out_specs=pl.BlockSpec((_MLP_BLOCK, HIDDEN_DIM), lambda i: (i, 0)),
        out_shape=jax.ShapeDtypeStruct((n, HIDDEN_DIM), jnp.float32),
    )(emb, W, b, gamma, beta)


def kernel(task_ids, table, W, b, gamma, beta):
    ids = task_ids.reshape(BATCH).astype(jnp.int32)
    ids_sc = ids[:SC_HALF].reshape(NUM_WORKERS, ROWS_PER_WORKER)
    ids_tc = ids[SC_HALF:]
    b2 = b.reshape(1, HIDDEN_DIM)
    g2 = gamma.reshape(1, HIDDEN_DIM)
    be2 = beta.reshape(1, HIDDEN_DIM)
    emb_sc = _sc_gather(table, ids_sc)
    out_tc = _tc_gather_mlp(ids_tc, table, W, b2, g2, be2)
    out_sc = _tc_mlp(emb_sc, W, b2, g2, be2)
    return jnp.concatenate([out_sc, out_tc], axis=0)


# speedup vs baseline: 1.3612x; 1.3612x over previous
import functools

import jax
import jax.numpy as jnp
from jax import lax
from jax.experimental import pallas as pl
from jax.experimental.pallas import tpu as pltpu
from jax.experimental.pallas import tpu_sc as plsc

BATCH = 16384
EMBED_DIM = 32
HIDDEN_DIM = 64
EPS = 1e-5

LANES = 128
PACK = LANES // EMBED_DIM
TABLE_ROWS = 1000001
_TP_BLOCK = 4096
_TP_NBLK = 62
WIDE_ROWS = _TP_NBLK * _TP_BLOCK

NUM_CORES = 2
NUM_SUBCORES = 16
NUM_WORKERS = NUM_CORES * NUM_SUBCORES
ROWS_PER_WORKER = BATCH // NUM_WORKERS
GATHER_CHUNK = 128
CHUNKS_PER_WORKER = ROWS_PER_WORKER // GATHER_CHUNK


def _tp_body(in0, in1, in2, in3, out_ref):
    out_ref[...] = jnp.concatenate(
        [in0[...].T, in1[...].T, in2[...].T, in3[...].T], axis=1
    )


def _tc_transpose_pack(tt):
    last_blk = (TABLE_ROWS - 1) // _TP_BLOCK
    spec = lambda a: pl.BlockSpec(
        (EMBED_DIM, _TP_BLOCK),
        lambda k, a=a: (0, jnp.minimum(a * _TP_NBLK + k, last_blk)),
    )
    return pl.pallas_call(
        _tp_body,
        grid=(_TP_NBLK,),
        in_specs=[spec(0), spec(1), spec(2), spec(3)],
        out_specs=pl.BlockSpec((_TP_BLOCK, LANES), lambda k: (k, 0)),
        out_shape=jax.ShapeDtypeStruct((WIDE_ROWS, LANES), jnp.float32),
    )(tt, tt, tt, tt)


def _sc_gather(lin, idx2d):
    mesh = plsc.VectorSubcoreMesh(core_axis_name="c", subcore_axis_name="s")

    @functools.partial(
        pl.kernel,
        mesh=mesh,
        out_type=jax.ShapeDtypeStruct((BATCH, LANES), jnp.float32),
        scratch_types=[
            pltpu.VMEM((CHUNKS_PER_WORKER, GATHER_CHUNK), jnp.int32),
            pltpu.VMEM((ROWS_PER_WORKER, LANES), jnp.float32),
            pltpu.SemaphoreType.DMA,
        ],
    )
    def gather_kernel(lin_hbm, idx_hbm, out_hbm, idx_v, rows_v, sem):
        wid = lax.axis_index("s") * NUM_CORES + lax.axis_index("c")
        pltpu.sync_copy(
            idx_hbm.at[pl.ds(wid * CHUNKS_PER_WORKER, CHUNKS_PER_WORKER)], idx_v
        )
        copies = []
        for j in range(CHUNKS_PER_WORKER):
            copies.append(
                pltpu.async_copy(
                    lin_hbm.at[idx_v.at[j]],
                    rows_v.at[pl.ds(j * GATHER_CHUNK, GATHER_CHUNK)],
                    sem,
                )
            )
        for c in copies:
            c.wait()
        pltpu.sync_copy(
            rows_v, out_hbm.at[pl.ds(wid * ROWS_PER_WORKER, ROWS_PER_WORKER)]
        )

    return gather_kernel(lin, idx2d)


_MLP_BLOCK = 2048


def _mlp_body(emb_ref, rmod_ref, w_ref, b_ref, g_ref, be_ref, out_ref):
    emb = emb_ref[...]
    group = lax.broadcasted_iota(jnp.int32, emb.shape, 1) // EMBED_DIM
    emb_sel = jnp.where(group == rmod_ref[...], emb, 0.0)
    h = lax.dot_general(
        emb_sel,
        w_ref[...],
        (((1,), (0,)), ((), ())),
        precision=lax.Precision.HIGHEST,
        preferred_element_type=jnp.float32,
    )
    h = h + b_ref[...]
    mu = jnp.mean(h, axis=1, keepdims=True)
    var = jnp.mean((h - mu) ** 2, axis=1, keepdims=True)
    hn = (h - mu) * lax.rsqrt(var + EPS)
    out_ref[...] = jnp.maximum(hn * g_ref[...] + be_ref[...], 0.0)


def _tc_mlp(emb, rmod, W_rep, b, gamma, beta):
    grid = (BATCH // _MLP_BLOCK,)
    return pl.pallas_call(
        _mlp_body,
        grid=grid,
        in_specs=[
            pl.BlockSpec((_MLP_BLOCK, LANES), lambda i: (i, 0)),
            pl.BlockSpec((_MLP_BLOCK, 1), lambda i: (i, 0)),
            pl.BlockSpec((LANES, HIDDEN_DIM), lambda i: (0, 0)),
            pl.BlockSpec((1, HIDDEN_DIM), lambda i: (0, 0)),
            pl.BlockSpec((1, HIDDEN_DIM), lambda i: (0, 0)),
            pl.BlockSpec((1, HIDDEN_DIM), lambda i: (0, 0)),
        ],
        out_specs=pl.BlockSpec((_MLP_BLOCK, HIDDEN_DIM), lambda i: (i, 0)),
        out_shape=jax.ShapeDtypeStruct((BATCH, HIDDEN_DIM), jnp.float32),
    )(emb, rmod, W_rep, b, gamma, beta)


def kernel(task_ids, table, W, b, gamma, beta):
    ids = task_ids.reshape(BATCH).astype(jnp.int32)
    lin = _tc_transpose_pack(table.T)
    idx2d = (ids % WIDE_ROWS).reshape(BATCH // GATHER_CHUNK, GATHER_CHUNK)
    rmod = (ids // WIDE_ROWS).reshape(BATCH, 1)
    emb128 = _sc_gather(lin, idx2d)
    W_rep = jnp.tile(W, (PACK, 1))
    return _tc_mlp(
        emb128,
        rmod,
        W_rep,
        b.reshape(1, HIDDEN_DIM),
        gamma.reshape(1, HIDDEN_DIM),
        beta.reshape(1, HIDDEN_DIM),
    )


# speedup vs baseline: 2.4960x; 1.8337x over previous
import functools

import jax
import jax.numpy as jnp
from jax import lax
from jax.experimental import pallas as pl
from jax.experimental.pallas import tpu as pltpu
from jax.experimental.pallas import tpu_sc as plsc

BATCH = 16384
EMBED_DIM = 32
HIDDEN_DIM = 64
EPS = 1e-5

LANES = 128
PACK = LANES // EMBED_DIM
TABLE_ROWS = 1000001
_TP_BLOCK = 4096
_TP_NBLK = 62
WIDE_ROWS = _TP_NBLK * _TP_BLOCK

NUM_CORES = 2
NUM_SUBCORES = 16
NUM_WORKERS = NUM_CORES * NUM_SUBCORES
ROWS_PER_WORKER = BATCH // NUM_WORKERS
GATHER_CHUNK = 128
CHUNKS_PER_WORKER = ROWS_PER_WORKER // GATHER_CHUNK


def _tp_body(in0, in1, in2, in3, out_ref):
    x = jnp.concatenate([in0[...], in1[...], in2[...], in3[...]], axis=0)
    out_ref[...] = x.T


def _tc_transpose_pack(tt):
    last_blk = (TABLE_ROWS - 1) // _TP_BLOCK
    spec = lambda a: pl.BlockSpec(
        (EMBED_DIM, _TP_BLOCK),
        lambda k, a=a: (0, jnp.minimum(a * _TP_NBLK + k, last_blk)),
    )
    return pl.pallas_call(
        _tp_body,
        grid=(_TP_NBLK,),
        in_specs=[spec(0), spec(1), spec(2), spec(3)],
        out_specs=pl.BlockSpec((_TP_BLOCK, LANES), lambda k: (k, 0)),
        out_shape=jax.ShapeDtypeStruct((WIDE_ROWS, LANES), jnp.float32),
    )(tt, tt, tt, tt)


def _sc_gather(lin, idx2d):
    mesh = plsc.VectorSubcoreMesh(core_axis_name="c", subcore_axis_name="s")

    @functools.partial(
        pl.kernel,
        mesh=mesh,
        out_type=jax.ShapeDtypeStruct((BATCH, LANES), jnp.float32),
        scratch_types=[
            pltpu.VMEM((CHUNKS_PER_WORKER, GATHER_CHUNK), jnp.int32),
            pltpu.VMEM((ROWS_PER_WORKER, LANES), jnp.float32),
            pltpu.SemaphoreType.DMA,
        ],
    )
    def gather_kernel(lin_hbm, idx_hbm, out_hbm, idx_v, rows_v, sem):
        wid = lax.axis_index("s") * NUM_CORES + lax.axis_index("c")
        pltpu.sync_copy(
            idx_hbm.at[pl.ds(wid * CHUNKS_PER_WORKER, CHUNKS_PER_WORKER)], idx_v
        )
        copies = []
        for j in range(CHUNKS_PER_WORKER):
            copies.append(
                pltpu.async_copy(
                    lin_hbm.at[idx_v.at[j]],
                    rows_v.at[pl.ds(j * GATHER_CHUNK, GATHER_CHUNK)],
                    sem,
                )
            )
        for c in copies:
            c.wait()
        pltpu.sync_copy(
            rows_v, out_hbm.at[pl.ds(wid * ROWS_PER_WORKER, ROWS_PER_WORKER)]
        )

    return gather_kernel(lin, idx2d)


_MLP_BLOCK = 2048


def _mlp_body(emb_ref, rmod_ref, w_ref, b_ref, g_ref, be_ref, out_ref):
    emb = emb_ref[...]
    group = lax.broadcasted_iota(jnp.int32, emb.shape, 1) // EMBED_DIM
    emb_sel = jnp.where(group == rmod_ref[...], emb, 0.0)
    h = lax.dot_general(
        emb_sel,
        w_ref[...],
        (((1,), (0,)), ((), ())),
        precision=lax.Precision.HIGHEST,
        preferred_element_type=jnp.float32,
    )
    h = h + b_ref[...]
    mu = jnp.mean(h, axis=1, keepdims=True)
    var = jnp.mean((h - mu) ** 2, axis=1, keepdims=True)
    hn = (h - mu) * lax.rsqrt(var + EPS)
    out_ref[...] = jnp.maximum(hn * g_ref[...] + be_ref[...], 0.0)


def _tc_mlp(emb, rmod, W_rep, b, gamma, beta):
    grid = (BATCH // _MLP_BLOCK,)
    return pl.pallas_call(
        _mlp_body,
        grid=grid,
        in_specs=[
            pl.BlockSpec((_MLP_BLOCK, LANES), lambda i: (i, 0)),
            pl.BlockSpec((_MLP_BLOCK, 1), lambda i: (i, 0)),
            pl.BlockSpec((LANES, HIDDEN_DIM), lambda i: (0, 0)),
            pl.BlockSpec((1, HIDDEN_DIM), lambda i: (0, 0)),
            pl.BlockSpec((1, HIDDEN_DIM), lambda i: (0, 0)),
            pl.BlockSpec((1, HIDDEN_DIM), lambda i: (0, 0)),
        ],
        out_specs=pl.BlockSpec((_MLP_BLOCK, HIDDEN_DIM), lambda i: (i, 0)),
        out_shape=jax.ShapeDtypeStruct((BATCH, HIDDEN_DIM), jnp.float32),
    )(emb, rmod, W_rep, b, gamma, beta)


def kernel(task_ids, table, W, b, gamma, beta):
    ids = task_ids.reshape(BATCH).astype(jnp.int32)
    lin = _tc_transpose_pack(table.T)
    idx2d = (ids % WIDE_ROWS).reshape(BATCH // GATHER_CHUNK, GATHER_CHUNK)
    rmod = (ids // WIDE_ROWS).reshape(BATCH, 1)
    emb128 = _sc_gather(lin, idx2d)
    W_rep = jnp.tile(W, (PACK, 1))
    return _tc_mlp(
        emb128,
        rmod,
        W_rep,
        b.reshape(1, HIDDEN_DIM),
        gamma.reshape(1, HIDDEN_DIM),
        beta.reshape(1, HIDDEN_DIM),
    )


# speedup vs baseline: 2.4985x; 1.0010x over previous
import functools

import jax
import jax.numpy as jnp
from jax import lax
from jax.experimental import pallas as pl
from jax.experimental.pallas import tpu as pltpu
from jax.experimental.pallas import tpu_sc as plsc

BATCH = 16384
EMBED_DIM = 32
HIDDEN_DIM = 64
EPS = 1e-5

LANES = 128
PACK = LANES // EMBED_DIM
TABLE_ROWS = 1000001
_TP_BLOCK = 4096
_TP_NBLK = 62
WIDE_ROWS = _TP_NBLK * _TP_BLOCK

NUM_CORES = 2
NUM_SUBCORES = 16
NUM_WORKERS = NUM_CORES * NUM_SUBCORES
ROWS_PER_WORKER = BATCH // NUM_WORKERS
GATHER_CHUNK = 128
CHUNKS_PER_WORKER = ROWS_PER_WORKER // GATHER_CHUNK


def _tp_body(in0, in1, in2, in3, out_ref):
    x = jnp.concatenate([in0[...], in1[...], in2[...], in3[...]], axis=0)
    for c in range(_TP_BLOCK // LANES):
        out_ref[pl.ds(c * LANES, LANES), :] = x[:, c * LANES : (c + 1) * LANES].T


def _tc_transpose_pack(tt):
    last_blk = (TABLE_ROWS - 1) // _TP_BLOCK
    spec = lambda a: pl.BlockSpec(
        (EMBED_DIM, _TP_BLOCK),
        lambda k, a=a: (0, jnp.minimum(a * _TP_NBLK + k, last_blk)),
    )
    return pl.pallas_call(
        _tp_body,
        grid=(_TP_NBLK,),
        in_specs=[spec(0), spec(1), spec(2), spec(3)],
        out_specs=pl.BlockSpec((_TP_BLOCK, LANES), lambda k: (k, 0)),
        out_shape=jax.ShapeDtypeStruct((WIDE_ROWS, LANES), jnp.float32),
    )(tt, tt, tt, tt)


def _sc_gather(lin, idx2d):
    mesh = plsc.VectorSubcoreMesh(core_axis_name="c", subcore_axis_name="s")

    @functools.partial(
        pl.kernel,
        mesh=mesh,
        out_type=jax.ShapeDtypeStruct((BATCH, LANES), jnp.float32),
        scratch_types=[
            pltpu.VMEM((CHUNKS_PER_WORKER, GATHER_CHUNK), jnp.int32),
            pltpu.VMEM((ROWS_PER_WORKER, LANES), jnp.float32),
            pltpu.SemaphoreType.DMA,
        ],
    )
    def gather_kernel(lin_hbm, idx_hbm, out_hbm, idx_v, rows_v, sem):
        wid = lax.axis_index("s") * NUM_CORES + lax.axis_index("c")
        pltpu.sync_copy(
            idx_hbm.at[pl.ds(wid * CHUNKS_PER_WORKER, CHUNKS_PER_WORKER)], idx_v
        )
        copies = []
        for j in range(CHUNKS_PER_WORKER):
            copies.append(
                pltpu.async_copy(
                    lin_hbm.at[idx_v.at[j]],
                    rows_v.at[pl.ds(j * GATHER_CHUNK, GATHER_CHUNK)],
                    sem,
                )
            )
        for c in copies:
            c.wait()
        pltpu.sync_copy(
            rows_v, out_hbm.at[pl.ds(wid * ROWS_PER_WORKER, ROWS_PER_WORKER)]
        )

    return gather_kernel(lin, idx2d)


_MLP_BLOCK = 2048


def _mlp_body(emb_ref, rmod_ref, w_ref, b_ref, g_ref, be_ref, out_ref):
    emb = emb_ref[...]
    group = lax.broadcasted_iota(jnp.int32, emb.shape, 1) // EMBED_DIM
    emb_sel = jnp.where(group == rmod_ref[...], emb, 0.0)
    h = lax.dot_general(
        emb_sel,
        w_ref[...],
        (((1,), (0,)), ((), ())),
        precision=lax.Precision.HIGHEST,
        preferred_element_type=jnp.float32,
    )
    h = h + b_ref[...]
    mu = jnp.mean(h, axis=1, keepdims=True)
    var = jnp.mean((h - mu) ** 2, axis=1, keepdims=True)
    hn = (h - mu) * lax.rsqrt(var + EPS)
    out_ref[...] = jnp.maximum(hn * g_ref[...] + be_ref[...], 0.0)


def _tc_mlp(emb, rmod, W_rep, b, gamma, beta):
    grid = (BATCH // _MLP_BLOCK,)
    return pl.pallas_call(
        _mlp_body,
        grid=grid,
        in_specs=[
            pl.BlockSpec((_MLP_BLOCK, LANES), lambda i: (i, 0)),
            pl.BlockSpec((_MLP_BLOCK, 1), lambda i: (i, 0)),
            pl.BlockSpec((LANES, HIDDEN_DIM), lambda i: (0, 0)),
            pl.BlockSpec((1, HIDDEN_DIM), lambda i: (0, 0)),
            pl.BlockSpec((1, HIDDEN_DIM), lambda i: (0, 0)),
            pl.BlockSpec((1, HIDDEN_DIM), lambda i: (0, 0)),
        ],
        out_specs=pl.BlockSpec((_MLP_BLOCK, HIDDEN_DIM), lambda i: (i, 0)),
        out_shape=jax.ShapeDtypeStruct((BATCH, HIDDEN_DIM), jnp.float32),
    )(emb, rmod, W_rep, b, gamma, beta)


def kernel(task_ids, table, W, b, gamma, beta):
    ids = task_ids.reshape(BATCH).astype(jnp.int32)
    lin = _tc_transpose_pack(table.T)
    idx2d = (ids % WIDE_ROWS).reshape(BATCH // GATHER_CHUNK, GATHER_CHUNK)
    rmod = (ids // WIDE_ROWS).reshape(BATCH, 1)
    emb128 = _sc_gather(lin, idx2d)
    W_rep = jnp.tile(W, (PACK, 1))
    return _tc_mlp(
        emb128,
        rmod,
        W_rep,
        b.reshape(1, HIDDEN_DIM),
        gamma.reshape(1, HIDDEN_DIM),
        beta.reshape(1, HIDDEN_DIM),
    )


# speedup vs baseline: 3.3719x; 1.3496x over previous
import functools

import jax
import jax.numpy as jnp
from jax import lax
from jax.experimental import pallas as pl
from jax.experimental.pallas import tpu as pltpu
from jax.experimental.pallas import tpu_sc as plsc

BATCH = 16384
EMBED_DIM = 32
HIDDEN_DIM = 64
EPS = 1e-5

LANES = 128
PACK = LANES // EMBED_DIM
TABLE_ROWS = 1000001
_TP_BLOCK = 8192
_TP_NBLK = 31
WIDE_ROWS = _TP_NBLK * _TP_BLOCK

NUM_CORES = 2
NUM_SUBCORES = 16
NUM_WORKERS = NUM_CORES * NUM_SUBCORES
ROWS_PER_WORKER = BATCH // NUM_WORKERS
GATHER_CHUNK = 128
CHUNKS_PER_WORKER = ROWS_PER_WORKER // GATHER_CHUNK
SC_LANES = 16


def _tp_body(in0, in1, in2, in3, out_ref):
    x = jnp.concatenate([in0[...], in1[...], in2[...], in3[...]], axis=0)
    for c in range(_TP_BLOCK // LANES):
        out_ref[pl.ds(c * LANES, LANES), :] = x[:, c * LANES : (c + 1) * LANES].T


def _tc_transpose_pack(tt):
    last_blk = (TABLE_ROWS - 1) // _TP_BLOCK
    spec = lambda a: pl.BlockSpec(
        (EMBED_DIM, _TP_BLOCK),
        lambda k, a=a: (0, jnp.minimum(a * _TP_NBLK + k, last_blk)),
    )
    return pl.pallas_call(
        _tp_body,
        grid=(_TP_NBLK,),
        in_specs=[spec(0), spec(1), spec(2), spec(3)],
        out_specs=pl.BlockSpec((_TP_BLOCK, LANES), lambda k: (k, 0)),
        out_shape=jax.ShapeDtypeStruct((WIDE_ROWS, LANES), jnp.float32),
    )(tt, tt, tt, tt)


def _sc_gather(lin, ids2d):
    mesh = plsc.VectorSubcoreMesh(core_axis_name="c", subcore_axis_name="s")

    @functools.partial(
        pl.kernel,
        mesh=mesh,
        out_type=jax.ShapeDtypeStruct((BATCH, LANES), jnp.float32),
        scratch_types=[
            pltpu.VMEM((CHUNKS_PER_WORKER, GATHER_CHUNK), jnp.int32),
            pltpu.VMEM((ROWS_PER_WORKER, LANES), jnp.float32),
            pltpu.SemaphoreType.DMA,
        ],
    )
    def gather_kernel(lin_hbm, idx_hbm, out_hbm, idx_v, rows_v, sem):
        wid = lax.axis_index("s") * NUM_CORES + lax.axis_index("c")
        pltpu.sync_copy(
            idx_hbm.at[pl.ds(wid * CHUNKS_PER_WORKER, CHUNKS_PER_WORKER)], idx_v
        )
        for j in range(CHUNKS_PER_WORKER):
            for t in range(GATHER_CHUNK // SC_LANES):
                sl = pl.ds(t * SC_LANES, SC_LANES)
                idx_v[j, sl] = lax.rem(idx_v[j, sl], WIDE_ROWS)
        copies = []
        for j in range(CHUNKS_PER_WORKER):
            copies.append(
                pltpu.async_copy(
                    lin_hbm.at[idx_v.at[j]],
                    rows_v.at[pl.ds(j * GATHER_CHUNK, GATHER_CHUNK)],
                    sem,
                )
            )
        for c in copies:
            c.wait()
        pltpu.sync_copy(
            rows_v, out_hbm.at[pl.ds(wid * ROWS_PER_WORKER, ROWS_PER_WORKER)]
        )

    return gather_kernel(lin, ids2d)


_MLP_BLOCK = 2048


def _mlp_body(emb_ref, ids_ref, w_ref, b_ref, g_ref, be_ref, out_ref):
    emb = emb_ref[...]
    rmod = ids_ref[...] // WIDE_ROWS
    group = lax.broadcasted_iota(jnp.int32, emb.shape, 1) // EMBED_DIM
    emb_sel = jnp.where(group == rmod, emb, 0.0)
    h = lax.dot_general(
        emb_sel,
        w_ref[...],
        (((1,), (0,)), ((), ())),
        preferred_element_type=jnp.float32,
    )
    h = h + b_ref[...]
    mu = jnp.mean(h, axis=1, keepdims=True)
    var = jnp.mean((h - mu) ** 2, axis=1, keepdims=True)
    hn = (h - mu) * lax.rsqrt(var + EPS)
    o = jnp.maximum(hn * g_ref[...] + be_ref[...], 0.0)
    for c in range(_MLP_BLOCK // LANES):
        out_ref[:, pl.ds(c * LANES, LANES)] = o[c * LANES : (c + 1) * LANES, :].T


def _tc_mlp_t(emb, ids, W_rep, b, gamma, beta):
    grid = (BATCH // _MLP_BLOCK,)
    return pl.pallas_call(
        _mlp_body,
        grid=grid,
        in_specs=[
            pl.BlockSpec((_MLP_BLOCK, LANES), lambda i: (i, 0)),
            pl.BlockSpec((_MLP_BLOCK, 1), lambda i: (i, 0)),
            pl.BlockSpec((LANES, HIDDEN_DIM), lambda i: (0, 0)),
            pl.BlockSpec((1, HIDDEN_DIM), lambda i: (0, 0)),
            pl.BlockSpec((1, HIDDEN_DIM), lambda i: (0, 0)),
            pl.BlockSpec((1, HIDDEN_DIM), lambda i: (0, 0)),
        ],
        out_specs=pl.BlockSpec((HIDDEN_DIM, _MLP_BLOCK), lambda i: (0, i)),
        out_shape=jax.ShapeDtypeStruct((HIDDEN_DIM, BATCH), jnp.float32),
    )(emb, ids, W_rep, b, gamma, beta)


def kernel(task_ids, table, W, b, gamma, beta):
    ids = task_ids.astype(jnp.int32)
    lin = _tc_transpose_pack(table.T)
    ids2d = ids.reshape(BATCH // GATHER_CHUNK, GATHER_CHUNK)
    emb128 = _sc_gather(lin, ids2d)
    W_rep = jnp.tile(W, (PACK, 1))
    out_t = _tc_mlp_t(
        emb128,
        ids.reshape(BATCH, 1),
        W_rep,
        b.reshape(1, HIDDEN_DIM),
        gamma.reshape(1, HIDDEN_DIM),
        beta.reshape(1, HIDDEN_DIM),
    )
    return out_t.T


# speedup vs baseline: 3.4862x; 1.0339x over previous
import functools

import jax
import jax.numpy as jnp
from jax import lax
from jax.experimental import pallas as pl
from jax.experimental.pallas import tpu as pltpu
from jax.experimental.pallas import tpu_sc as plsc

BATCH = 16384
EMBED_DIM = 32
HIDDEN_DIM = 64
EPS = 1e-5

LANES = 128
PACK = LANES // EMBED_DIM
TABLE_ROWS = 1000001
_TP_BLOCK = 16384
_TP_NBLK = 16
WIDE_ROWS = _TP_NBLK * _TP_BLOCK

NUM_CORES = 2
NUM_SUBCORES = 16
NUM_WORKERS = NUM_CORES * NUM_SUBCORES
ROWS_PER_WORKER = BATCH // NUM_WORKERS
GATHER_CHUNK = 128
CHUNKS_PER_WORKER = ROWS_PER_WORKER // GATHER_CHUNK
SC_LANES = 16


def _tp_body(in0, in1, in2, in3, out_ref):
    x = jnp.concatenate([in0[...], in1[...], in2[...], in3[...]], axis=0)
    for c in range(_TP_BLOCK // LANES):
        out_ref[pl.ds(c * LANES, LANES), :] = x[:, c * LANES : (c + 1) * LANES].T


def _tc_transpose_pack(tt):
    last_blk = (TABLE_ROWS - 1) // _TP_BLOCK
    spec = lambda a: pl.BlockSpec(
        (EMBED_DIM, _TP_BLOCK),
        lambda k, a=a: (0, jnp.minimum(a * _TP_NBLK + k, last_blk)),
    )
    return pl.pallas_call(
        _tp_body,
        grid=(_TP_NBLK,),
        in_specs=[spec(0), spec(1), spec(2), spec(3)],
        out_specs=pl.BlockSpec((_TP_BLOCK, LANES), lambda k: (k, 0)),
        out_shape=jax.ShapeDtypeStruct((WIDE_ROWS, LANES), jnp.float32),
    )(tt, tt, tt, tt)


def _sc_gather(lin, ids2d):
    mesh = plsc.VectorSubcoreMesh(core_axis_name="c", subcore_axis_name="s")

    @functools.partial(
        pl.kernel,
        mesh=mesh,
        out_type=jax.ShapeDtypeStruct((BATCH, LANES), jnp.float32),
        scratch_types=[
            pltpu.VMEM((CHUNKS_PER_WORKER, GATHER_CHUNK), jnp.int32),
            pltpu.VMEM((ROWS_PER_WORKER, LANES), jnp.float32),
            pltpu.SemaphoreType.DMA,
        ],
    )
    def gather_kernel(lin_hbm, idx_hbm, out_hbm, idx_v, rows_v, sem):
        wid = lax.axis_index("s") * NUM_CORES + lax.axis_index("c")
        pltpu.sync_copy(
            idx_hbm.at[pl.ds(wid * CHUNKS_PER_WORKER, CHUNKS_PER_WORKER)], idx_v
        )
        for j in range(CHUNKS_PER_WORKER):
            for t in range(GATHER_CHUNK // SC_LANES):
                sl = pl.ds(t * SC_LANES, SC_LANES)
                idx_v[j, sl] = lax.rem(idx_v[j, sl], WIDE_ROWS)
        copies = []
        for j in range(CHUNKS_PER_WORKER):
            copies.append(
                pltpu.async_copy(
                    lin_hbm.at[idx_v.at[j]],
                    rows_v.at[pl.ds(j * GATHER_CHUNK, GATHER_CHUNK)],
                    sem,
                )
            )
        for c in copies:
            c.wait()
        pltpu.sync_copy(
            rows_v, out_hbm.at[pl.ds(wid * ROWS_PER_WORKER, ROWS_PER_WORKER)]
        )

    return gather_kernel(lin, ids2d)


_MLP_BLOCK = 2048


def _mlp_body(emb_ref, ids_ref, w_ref, b_ref, g_ref, be_ref, out_ref):
    emb = emb_ref[...]
    rmod = ids_ref[...] // WIDE_ROWS
    group = lax.broadcasted_iota(jnp.int32, emb.shape, 1) // EMBED_DIM
    emb_sel = jnp.where(group == rmod, emb, 0.0)
    h = lax.dot_general(
        emb_sel,
        w_ref[...],
        (((1,), (0,)), ((), ())),
        preferred_element_type=jnp.float32,
    )
    h = h + b_ref[...]
    mu = jnp.mean(h, axis=1, keepdims=True)
    var = jnp.mean((h - mu) ** 2, axis=1, keepdims=True)
    hn = (h - mu) * lax.rsqrt(var + EPS)
    o = jnp.maximum(hn * g_ref[...] + be_ref[...], 0.0)
    for c in range(_MLP_BLOCK // LANES):
        out_ref[:, pl.ds(c * LANES, LANES)] = o[c * LANES : (c + 1) * LANES, :].T


def _tc_mlp_t(emb, ids, W_rep, b, gamma, beta):
    grid = (BATCH // _MLP_BLOCK,)
    return pl.pallas_call(
        _mlp_body,
        grid=grid,
        in_specs=[
            pl.BlockSpec((_MLP_BLOCK, LANES), lambda i: (i, 0)),
            pl.BlockSpec((_MLP_BLOCK, 1), lambda i: (i, 0)),
            pl.BlockSpec((LANES, HIDDEN_DIM), lambda i: (0, 0)),
            pl.BlockSpec((1, HIDDEN_DIM), lambda i: (0, 0)),
            pl.BlockSpec((1, HIDDEN_DIM), lambda i: (0, 0)),
            pl.BlockSpec((1, HIDDEN_DIM), lambda i: (0, 0)),
        ],
        out_specs=pl.BlockSpec((HIDDEN_DIM, _MLP_BLOCK), lambda i: (0, i)),
        out_shape=jax.ShapeDtypeStruct((HIDDEN_DIM, BATCH), jnp.float32),
    )(emb, ids, W_rep, b, gamma, beta)


def kernel(task_ids, table, W, b, gamma, beta):
    ids = task_ids.astype(jnp.int32)
    lin = _tc_transpose_pack(table.T)
    ids2d = ids.reshape(BATCH // GATHER_CHUNK, GATHER_CHUNK)
    emb128 = _sc_gather(lin, ids2d)
    W_rep = jnp.tile(W, (PACK, 1))
    out_t = _tc_mlp_t(
        emb128,
        ids.reshape(BATCH, 1),
        W_rep,
        b.reshape(1, HIDDEN_DIM),
        gamma.reshape(1, HIDDEN_DIM),
        beta.reshape(1, HIDDEN_DIM),
    )
    return out_t.T


# speedup vs baseline: 3.4935x; 1.0021x over previous
import functools

import jax
import jax.numpy as jnp
from jax import lax
from jax.experimental import pallas as pl
from jax.experimental.pallas import tpu as pltpu
from jax.experimental.pallas import tpu_sc as plsc

BATCH = 16384
EMBED_DIM = 32
HIDDEN_DIM = 64
EPS = 1e-5

LANES = 128
PACK = LANES // EMBED_DIM
TABLE_ROWS = 1000001
_TP_BLOCK = 16384
_TP_NBLK = 16
WIDE_ROWS = _TP_NBLK * _TP_BLOCK

NUM_CORES = 2
NUM_SUBCORES = 16
NUM_WORKERS = NUM_CORES * NUM_SUBCORES
ROWS_PER_WORKER = BATCH // NUM_WORKERS
GATHER_CHUNK = 128
CHUNKS_PER_WORKER = ROWS_PER_WORKER // GATHER_CHUNK
SC_LANES = 16


def _tp_body(in0, in1, in2, in3, out_ref):
    x = jnp.concatenate([in0[...], in1[...], in2[...], in3[...]], axis=0)
    for c in range(_TP_BLOCK // LANES):
        out_ref[pl.ds(c * LANES, LANES), :] = x[:, c * LANES : (c + 1) * LANES].T


def _tc_transpose_pack(tt):
    last_blk = (TABLE_ROWS - 1) // _TP_BLOCK
    spec = lambda a: pl.BlockSpec(
        (EMBED_DIM, _TP_BLOCK),
        lambda k, a=a: (0, jnp.minimum(a * _TP_NBLK + k, last_blk)),
    )
    return pl.pallas_call(
        _tp_body,
        grid=(_TP_NBLK,),
        in_specs=[spec(0), spec(1), spec(2), spec(3)],
        out_specs=pl.BlockSpec((_TP_BLOCK, LANES), lambda k: (k, 0)),
        out_shape=jax.ShapeDtypeStruct((WIDE_ROWS, LANES), jnp.float32),
    )(tt, tt, tt, tt)


def _sc_gather(lin, ids2d):
    mesh = plsc.VectorSubcoreMesh(core_axis_name="c", subcore_axis_name="s")

    @functools.partial(
        pl.kernel,
        mesh=mesh,
        out_type=jax.ShapeDtypeStruct((BATCH, LANES), jnp.float32),
        scratch_types=[
            pltpu.VMEM((CHUNKS_PER_WORKER, GATHER_CHUNK), jnp.int32),
            pltpu.VMEM((ROWS_PER_WORKER, LANES), jnp.float32),
            pltpu.SemaphoreType.DMA,
        ],
    )
    def gather_kernel(lin_hbm, idx_hbm, out_hbm, idx_v, rows_v, sem):
        wid = lax.axis_index("s") * NUM_CORES + lax.axis_index("c")
        pltpu.sync_copy(
            idx_hbm.at[pl.ds(wid * CHUNKS_PER_WORKER, CHUNKS_PER_WORKER)], idx_v
        )
        for j in range(CHUNKS_PER_WORKER):
            for t in range(GATHER_CHUNK // SC_LANES):
                sl = pl.ds(t * SC_LANES, SC_LANES)
                idx_v[j, sl] = lax.rem(idx_v[j, sl], WIDE_ROWS)
        copies = []
        for j in range(CHUNKS_PER_WORKER):
            copies.append(
                pltpu.async_copy(
                    lin_hbm.at[idx_v.at[j]],
                    rows_v.at[pl.ds(j * GATHER_CHUNK, GATHER_CHUNK)],
                    sem,
                )
            )
        for c in copies:
            c.wait()
        pltpu.sync_copy(
            rows_v, out_hbm.at[pl.ds(wid * ROWS_PER_WORKER, ROWS_PER_WORKER)]
        )

    return gather_kernel(lin, ids2d)


_MLP_BLOCK = 4096


def _mlp_body(emb_ref, ids_ref, w_ref, b_ref, g_ref, be_ref, out_ref):
    emb = emb_ref[...]
    rmod = ids_ref[...] // WIDE_ROWS
    group = lax.broadcasted_iota(jnp.int32, emb.shape, 1) // EMBED_DIM
    emb_sel = jnp.where(group == rmod, emb, 0.0)
    h = lax.dot_general(
        emb_sel,
        w_ref[...],
        (((1,), (0,)), ((), ())),
        preferred_element_type=jnp.float32,
    )
    h = h + b_ref[...]
    mu = jnp.mean(h, axis=1, keepdims=True)
    var = jnp.mean((h - mu) ** 2, axis=1, keepdims=True)
    hn = (h - mu) * lax.rsqrt(var + EPS)
    o = jnp.maximum(hn * g_ref[...] + be_ref[...], 0.0)
    for c in range(_MLP_BLOCK // LANES):
        out_ref[:, pl.ds(c * LANES, LANES)] = o[c * LANES : (c + 1) * LANES, :].T


def _tc_mlp_t(emb, ids, W_rep, b, gamma, beta):
    grid = (BATCH // _MLP_BLOCK,)
    return pl.pallas_call(
        _mlp_body,
        grid=grid,
        in_specs=[
            pl.BlockSpec((_MLP_BLOCK, LANES), lambda i: (i, 0)),
            pl.BlockSpec((_MLP_BLOCK, 1), lambda i: (i, 0)),
            pl.BlockSpec((LANES, HIDDEN_DIM), lambda i: (0, 0)),
            pl.BlockSpec((1, HIDDEN_DIM), lambda i: (0, 0)),
            pl.BlockSpec((1, HIDDEN_DIM), lambda i: (0, 0)),
            pl.BlockSpec((1, HIDDEN_DIM), lambda i: (0, 0)),
        ],
        out_specs=pl.BlockSpec((HIDDEN_DIM, _MLP_BLOCK), lambda i: (0, i)),
        out_shape=jax.ShapeDtypeStruct((HIDDEN_DIM, BATCH), jnp.float32),
    )(emb, ids, W_rep, b, gamma, beta)


def kernel(task_ids, table, W, b, gamma, beta):
    ids = task_ids.astype(jnp.int32)
    lin = _tc_transpose_pack(table.T)
    ids2d = ids.reshape(BATCH // GATHER_CHUNK, GATHER_CHUNK)
    emb128 = _sc_gather(lin, ids2d)
    W_rep = jnp.tile(W, (PACK, 1))
    out_t = _tc_mlp_t(
        emb128,
        ids.reshape(BATCH, 1),
        W_rep,
        b.reshape(1, HIDDEN_DIM),
        gamma.reshape(1, HIDDEN_DIM),
        beta.reshape(1, HIDDEN_DIM),
    )
    return out_t.T
